# baseline (speedup 1.0000x reference)
import jax
import jax.numpy as jnp
from jax import lax
from jax.experimental import pallas as pl
from jax.experimental.pallas import tpu as pltpu
from jax.experimental.pallas import tpu_sc as plsc

NN = 10000
NE = 160000
DI = 256
DH = 128
NC = 2
NS = 16
L = 16

CH = 32
CPT = 320
IBK = 8
NIB = CPT // IBK
EPT = CPT * CH
E_PAD = NS * EPT
N_PAD = 10240
ZROWS = N_PAD // NS
ZSTARTS = tuple(range(0, ZROWS, CH))

RB = 1000


def _sc_mesh():
  return plsc.VectorSubcoreMesh(core_axis_name="c", subcore_axis_name="s")


def _edge_pipeline(p_hbm, srci, dsti, s, srcblk, dstblk, rowsa, rowsb,
                   acc, sema, semb):
  def blk(b, carry):
    pltpu.sync_copy(srci.at[s, pl.ds(b * IBK, IBK)], srcblk)
    pltpu.sync_copy(dsti.at[s, pl.ds(b * IBK, IBK)], dstblk)
    cpa = pltpu.async_copy(p_hbm.at[srcblk.at[0]], rowsa, sema)
    for k in range(0, IBK - 2, 2):
      cpb = pltpu.async_copy(p_hbm.at[srcblk.at[k + 1]], rowsb, semb)
      cpa.wait()
      pltpu.sync_copy(rowsa, acc.at[dstblk.at[k]], add=True)
      cpa = pltpu.async_copy(p_hbm.at[srcblk.at[k + 2]], rowsa, sema)
      cpb.wait()
      pltpu.sync_copy(rowsb, acc.at[dstblk.at[k + 1]], add=True)
    cpb = pltpu.async_copy(p_hbm.at[srcblk.at[IBK - 1]], rowsb, semb)
    cpa.wait()
    pltpu.sync_copy(rowsa, acc.at[dstblk.at[IBK - 2]], add=True)
    cpb.wait()
    pltpu.sync_copy(rowsb, acc.at[dstblk.at[IBK - 1]], add=True)
    return carry
  lax.fori_loop(0, NIB, blk, 0)


def _agg_body(pa, pb, srci, dsti, zbig, agga, aggb,
              srcblk, dstblk, rowsa, rowsb, acc, sema, semb):
  c = lax.axis_index("c")
  s = lax.axis_index("s")

  pltpu.sync_copy(zbig, rowsa)
  for off in ZSTARTS:
    pltpu.sync_copy(rowsa, acc.at[pl.ds(s * ZROWS + off, CH)])
  plsc.subcore_barrier()

  @pl.when(c == 0)
  def _():
    _edge_pipeline(pa, srci, dsti, s, srcblk, dstblk, rowsa, rowsb, acc,
                   sema, semb)

  @pl.when(c == 1)
  def _():
    _edge_pipeline(pb, srci, dsti, s, srcblk, dstblk, rowsa, rowsb, acc,
                   sema, semb)

  plsc.subcore_barrier()

  @pl.when(c == 0)
  def _():
    for off in ZSTARTS:
      base = s * ZROWS + off
      pltpu.sync_copy(acc.at[pl.ds(base, CH)], rowsa)
      pltpu.sync_copy(rowsa, agga.at[pl.ds(base, CH)])

  @pl.when(c == 1)
  def _():
    for off in ZSTARTS:
      base = s * ZROWS + off
      pltpu.sync_copy(acc.at[pl.ds(base, CH)], rowsa)
      pltpu.sync_copy(rowsa, aggb.at[pl.ds(base, CH)])


_agg = pl.kernel(
    _agg_body,
    out_type=(
        jax.ShapeDtypeStruct((N_PAD, DH), jnp.float32),
        jax.ShapeDtypeStruct((N_PAD, DH), jnp.float32),
    ),
    mesh=_sc_mesh(),
    scratch_types=[
        pltpu.VMEM((IBK, CH), jnp.int32),
        pltpu.VMEM((IBK, CH), jnp.int32),
        pltpu.VMEM((CH, DH), jnp.float32),
        pltpu.VMEM((CH, DH), jnp.float32),
        pltpu.VMEM_SHARED((N_PAD, DH), jnp.float32),
        pltpu.SemaphoreType.DMA,
        pltpu.SemaphoreType.DMA,
    ],
)


CPT0 = CPT // 2


def _cnt_body(dsti, zbig, ones2, cnta, cntb, dstblk, buf, acc, sem):
  c = lax.axis_index("c")
  s = lax.axis_index("s")

  pltpu.sync_copy(zbig, buf)
  for off in ZSTARTS:
    pltpu.sync_copy(buf, acc.at[pl.ds(s * ZROWS + off, CH)])
  pltpu.sync_copy(ones2, buf)
  plsc.subcore_barrier()

  def blk(b, carry):
    pltpu.sync_copy(dsti.at[s, pl.ds(b * IBK, IBK)], dstblk)
    for k in range(IBK):
      pltpu.sync_copy(buf, acc.at[dstblk.at[k]], add=True)
    return carry

  @pl.when(c == 0)
  def _():
    lax.fori_loop(0, CPT0 // IBK, blk, 0)

  @pl.when(c == 1)
  def _():
    lax.fori_loop(CPT0 // IBK, NIB, blk, 0)

  plsc.subcore_barrier()

  @pl.when(c == 0)
  def _():
    for off in ZSTARTS:
      base = s * ZROWS + off
      pltpu.sync_copy(acc.at[pl.ds(base, CH)], buf)
      pltpu.sync_copy(buf, cnta.at[pl.ds(base, CH)])

  @pl.when(c == 1)
  def _():
    for off in ZSTARTS:
      base = s * ZROWS + off
      pltpu.sync_copy(acc.at[pl.ds(base, CH)], buf)
      pltpu.sync_copy(buf, cntb.at[pl.ds(base, CH)])


_cnt = pl.kernel(
    _cnt_body,
    out_type=(
        jax.ShapeDtypeStruct((N_PAD, DH), jnp.float32),
        jax.ShapeDtypeStruct((N_PAD, DH), jnp.float32),
    ),
    mesh=_sc_mesh(),
    scratch_types=[
        pltpu.VMEM((IBK, CH), jnp.int32),
        pltpu.VMEM((CH, DH), jnp.float32),
        pltpu.VMEM_SHARED((N_PAD, DH), jnp.float32),
        pltpu.SemaphoreType.DMA,
    ],
)


def _pre_body(x_ref, wl_ref, wr_ref, b_ref, pa_ref, pb_ref, s_ref):
  xb = x_ref[:]
  p = jnp.dot(xb, wl_ref[:], preferred_element_type=jnp.float32)
  pa_ref[:] = p[:, :DH]
  pb_ref[:] = p[:, DH:]
  s_ref[:] = jnp.dot(xb, wr_ref[:], preferred_element_type=jnp.float32) + b_ref[:]


_pre = pl.pallas_call(
    _pre_body,
    grid=(NN // RB,),
    in_specs=[
        pl.BlockSpec((RB, DI), lambda i: (i, 0)),
        pl.BlockSpec((DI, DI), lambda i: (0, 0)),
        pl.BlockSpec((DI, DI), lambda i: (0, 0)),
        pl.BlockSpec((1, DI), lambda i: (0, 0)),
    ],
    out_specs=[
        pl.BlockSpec((RB, DH), lambda i: (i, 0)),
        pl.BlockSpec((RB, DH), lambda i: (i, 0)),
        pl.BlockSpec((RB, DI), lambda i: (i, 0)),
    ],
    out_shape=[
        jax.ShapeDtypeStruct((NN, DH), jnp.float32),
        jax.ShapeDtypeStruct((NN, DH), jnp.float32),
        jax.ShapeDtypeStruct((NN, DI), jnp.float32),
    ],
)


def _mid_body(aa_ref, ab_ref, ca_ref, cb_ref, s1_ref, wl_ref, wr_ref, b_ref,
              pa_ref, pb_ref, s_ref):
  r = 1.0 / jnp.maximum(ca_ref[:, 0:1] + cb_ref[:, 0:1], 1.0)
  h = jnp.concatenate([aa_ref[:] * r, ab_ref[:] * r], axis=1) + s1_ref[:]
  h = jnp.maximum(h, 0.0)
  p = jnp.dot(h, wl_ref[:], preferred_element_type=jnp.float32)
  pa_ref[:] = p[:, :DH]
  pb_ref[:] = p[:, DH:]
  s_ref[:] = jnp.dot(h, wr_ref[:], preferred_element_type=jnp.float32) + b_ref[:]


_mid = pl.pallas_call(
    _mid_body,
    grid=(NN // RB,),
    in_specs=[
        pl.BlockSpec((RB, DH), lambda i: (i, 0)),
        pl.BlockSpec((RB, DH), lambda i: (i, 0)),
        pl.BlockSpec((RB, DH), lambda i: (i, 0)),
        pl.BlockSpec((RB, DH), lambda i: (i, 0)),
        pl.BlockSpec((RB, DI), lambda i: (i, 0)),
        pl.BlockSpec((DI, DI), lambda i: (0, 0)),
        pl.BlockSpec((DI, DI), lambda i: (0, 0)),
        pl.BlockSpec((1, DI), lambda i: (0, 0)),
    ],
    out_specs=[
        pl.BlockSpec((RB, DH), lambda i: (i, 0)),
        pl.BlockSpec((RB, DH), lambda i: (i, 0)),
        pl.BlockSpec((RB, DI), lambda i: (i, 0)),
    ],
    out_shape=[
        jax.ShapeDtypeStruct((NN, DH), jnp.float32),
        jax.ShapeDtypeStruct((NN, DH), jnp.float32),
        jax.ShapeDtypeStruct((NN, DI), jnp.float32),
    ],
)


def _post_body(aa_ref, ab_ref, ca_ref, cb_ref, s2_ref, wlin_ref, blin_ref,
               out_ref):
  r = 1.0 / jnp.maximum(ca_ref[:, 0:1] + cb_ref[:, 0:1], 1.0)
  h = jnp.concatenate([aa_ref[:] * r, ab_ref[:] * r], axis=1) + s2_ref[:]
  h = jnp.maximum(h, 0.0)
  out_ref[:] = (jnp.dot(h, wlin_ref[:], preferred_element_type=jnp.float32)
                + blin_ref[:])


_post = pl.pallas_call(
    _post_body,
    grid=(NN // RB,),
    in_specs=[
        pl.BlockSpec((RB, DH), lambda i: (i, 0)),
        pl.BlockSpec((RB, DH), lambda i: (i, 0)),
        pl.BlockSpec((RB, DH), lambda i: (i, 0)),
        pl.BlockSpec((RB, DH), lambda i: (i, 0)),
        pl.BlockSpec((RB, DI), lambda i: (i, 0)),
        pl.BlockSpec((DI, DI), lambda i: (0, 0)),
        pl.BlockSpec((1, DI), lambda i: (0, 0)),
    ],
    out_specs=pl.BlockSpec((RB, DI), lambda i: (i, 0)),
    out_shape=jax.ShapeDtypeStruct((NN, DI), jnp.float32),
)


@jax.jit
def kernel(x, edge_index, W1l, b1l, W1r, W2l, b2l, W2r, Wlin, blin):
  src = edge_index[0].astype(jnp.int32)
  dst = edge_index[1].astype(jnp.int32)
  srci = jnp.concatenate(
      [src, jnp.zeros((E_PAD - NE,), jnp.int32)]).reshape(NS, CPT, CH)
  dsti = jnp.concatenate(
      [dst, jnp.full((E_PAD - NE,), NN, jnp.int32)]).reshape(NS, CPT, CH)
  zbig = jnp.zeros((CH, DH), jnp.float32)
  ones2 = jnp.ones((CH, DH), jnp.float32)
  b1 = b1l.reshape(1, DI)
  b2 = b2l.reshape(1, DI)
  bl = blin.reshape(1, DI)

  pa, pb, s1 = _pre(x, W1l, W1r, b1)
  cnta, cntb = _cnt(dsti, zbig, ones2)
  agga, aggb = _agg(pa, pb, srci, dsti, zbig)
  p2a, p2b, s2 = _mid(agga[:NN], aggb[:NN], cnta[:NN], cntb[:NN], s1,
                      W2l, W2r, b2)
  a2a, a2b = _agg(p2a, p2b, srci, dsti, zbig)
  return _post(a2a[:NN], a2b[:NN], cnta[:NN], cntb[:NN], s2, Wlin, bl)


# speedup vs baseline: 1.2101x; 1.2101x over previous
import jax
import jax.numpy as jnp
from jax import lax
from jax.experimental import pallas as pl
from jax.experimental.pallas import tpu as pltpu
from jax.experimental.pallas import tpu_sc as plsc

NN = 10000
NE = 160000
DI = 256
DH = 128
NC = 2
NS = 16
L = 16

CH = 32
CPT = 320
IBK = 8
NIB = CPT // IBK
GB = 32
NBODY = CPT // GB
EPT = CPT * CH
E_PAD = NS * EPT
N_PAD = 10240
ZROWS = N_PAD // NS
ZSTARTS = tuple(range(0, ZROWS, CH))

RB = 1000


def _sc_mesh():
  return plsc.VectorSubcoreMesh(core_axis_name="c", subcore_axis_name="s")


def _edge_pipeline(p_hbm, srci, dsti, s, srcblk, dstblk, rows, sg, ss, acc):
  def blk(b, carry):
    pltpu.sync_copy(srci.at[s, pl.ds(b * GB, GB)], srcblk)
    pltpu.sync_copy(dsti.at[s, pl.ds(b * GB, GB)], dstblk)
    sgh = [None] * 4
    ssh = [None] * 4
    for j in range(GB):
      k = j % 4
      if ssh[k] is not None:
        ssh[k].wait()
      sgh[k] = pltpu.async_copy(p_hbm.at[srcblk.at[j]], rows[k], sg[k])
      if j >= 2:
        k2 = (j - 2) % 4
        sgh[k2].wait()
        ssh[k2] = pltpu.async_copy(rows[k2], acc.at[dstblk.at[j - 2]],
                                   ss[k2], add=True)
    for j in (GB - 2, GB - 1):
      k = j % 4
      sgh[k].wait()
      ssh[k] = pltpu.async_copy(rows[k], acc.at[dstblk.at[j]], ss[k],
                                add=True)
    for k in range(4):
      ssh[k].wait()
    return carry
  lax.fori_loop(0, NBODY, blk, 0)


def _agg_body(pa, pb, srci, dsti, zbig, agga, aggb,
              srcblk, dstblk, rows0, rows1, rows2, rows3, acc,
              sg0, sg1, sg2, sg3, ss0, ss1, ss2, ss3):
  c = lax.axis_index("c")
  s = lax.axis_index("s")
  rows = (rows0, rows1, rows2, rows3)
  sg = (sg0, sg1, sg2, sg3)
  ss = (ss0, ss1, ss2, ss3)

  pltpu.sync_copy(zbig, rows0)
  for off in ZSTARTS:
    pltpu.sync_copy(rows0, acc.at[pl.ds(s * ZROWS + off, CH)])
  plsc.subcore_barrier()

  @pl.when(c == 0)
  def _():
    _edge_pipeline(pa, srci, dsti, s, srcblk, dstblk, rows, sg, ss, acc)

  @pl.when(c == 1)
  def _():
    _edge_pipeline(pb, srci, dsti, s, srcblk, dstblk, rows, sg, ss, acc)

  plsc.subcore_barrier()

  @pl.when(c == 0)
  def _():
    for off in ZSTARTS:
      base = s * ZROWS + off
      pltpu.sync_copy(acc.at[pl.ds(base, CH)], rows0)
      pltpu.sync_copy(rows0, agga.at[pl.ds(base, CH)])

  @pl.when(c == 1)
  def _():
    for off in ZSTARTS:
      base = s * ZROWS + off
      pltpu.sync_copy(acc.at[pl.ds(base, CH)], rows0)
      pltpu.sync_copy(rows0, aggb.at[pl.ds(base, CH)])


_agg = pl.kernel(
    _agg_body,
    out_type=(
        jax.ShapeDtypeStruct((N_PAD, DH), jnp.float32),
        jax.ShapeDtypeStruct((N_PAD, DH), jnp.float32),
    ),
    mesh=_sc_mesh(),
    scratch_types=[
        pltpu.VMEM((GB, CH), jnp.int32),
        pltpu.VMEM((GB, CH), jnp.int32),
        pltpu.VMEM((CH, DH), jnp.float32),
        pltpu.VMEM((CH, DH), jnp.float32),
        pltpu.VMEM((CH, DH), jnp.float32),
        pltpu.VMEM((CH, DH), jnp.float32),
        pltpu.VMEM_SHARED((N_PAD, DH), jnp.float32),
        pltpu.SemaphoreType.DMA,
        pltpu.SemaphoreType.DMA,
        pltpu.SemaphoreType.DMA,
        pltpu.SemaphoreType.DMA,
        pltpu.SemaphoreType.DMA,
        pltpu.SemaphoreType.DMA,
        pltpu.SemaphoreType.DMA,
        pltpu.SemaphoreType.DMA,
    ],
)


CPT0 = CPT // 2


def _cnt_body(dsti, zbig, ones2, cnta, cntb, dstblk, buf, acc, sem):
  c = lax.axis_index("c")
  s = lax.axis_index("s")

  pltpu.sync_copy(zbig, buf)
  for off in ZSTARTS:
    pltpu.sync_copy(buf, acc.at[pl.ds(s * ZROWS + off, CH)])
  pltpu.sync_copy(ones2, buf)
  plsc.subcore_barrier()

  def blk(b, carry):
    pltpu.sync_copy(dsti.at[s, pl.ds(b * IBK, IBK)], dstblk)
    for k in range(IBK):
      pltpu.sync_copy(buf, acc.at[dstblk.at[k]], add=True)
    return carry

  @pl.when(c == 0)
  def _():
    lax.fori_loop(0, CPT0 // IBK, blk, 0)

  @pl.when(c == 1)
  def _():
    lax.fori_loop(CPT0 // IBK, NIB, blk, 0)

  plsc.subcore_barrier()

  @pl.when(c == 0)
  def _():
    for off in ZSTARTS:
      base = s * ZROWS + off
      pltpu.sync_copy(acc.at[pl.ds(base, CH)], buf)
      pltpu.sync_copy(buf, cnta.at[pl.ds(base, CH)])

  @pl.when(c == 1)
  def _():
    for off in ZSTARTS:
      base = s * ZROWS + off
      pltpu.sync_copy(acc.at[pl.ds(base, CH)], buf)
      pltpu.sync_copy(buf, cntb.at[pl.ds(base, CH)])


_cnt = pl.kernel(
    _cnt_body,
    out_type=(
        jax.ShapeDtypeStruct((N_PAD, DH), jnp.float32),
        jax.ShapeDtypeStruct((N_PAD, DH), jnp.float32),
    ),
    mesh=_sc_mesh(),
    scratch_types=[
        pltpu.VMEM((IBK, CH), jnp.int32),
        pltpu.VMEM((CH, DH), jnp.float32),
        pltpu.VMEM_SHARED((N_PAD, DH), jnp.float32),
        pltpu.SemaphoreType.DMA,
    ],
)


def _pre_body(x_ref, wl_ref, wr_ref, b_ref, pa_ref, pb_ref, s_ref):
  xb = x_ref[:]
  p = jnp.dot(xb, wl_ref[:], preferred_element_type=jnp.float32)
  pa_ref[:] = p[:, :DH]
  pb_ref[:] = p[:, DH:]
  s_ref[:] = jnp.dot(xb, wr_ref[:], preferred_element_type=jnp.float32) + b_ref[:]


_pre = pl.pallas_call(
    _pre_body,
    grid=(NN // RB,),
    in_specs=[
        pl.BlockSpec((RB, DI), lambda i: (i, 0)),
        pl.BlockSpec((DI, DI), lambda i: (0, 0)),
        pl.BlockSpec((DI, DI), lambda i: (0, 0)),
        pl.BlockSpec((1, DI), lambda i: (0, 0)),
    ],
    out_specs=[
        pl.BlockSpec((RB, DH), lambda i: (i, 0)),
        pl.BlockSpec((RB, DH), lambda i: (i, 0)),
        pl.BlockSpec((RB, DI), lambda i: (i, 0)),
    ],
    out_shape=[
        jax.ShapeDtypeStruct((NN, DH), jnp.float32),
        jax.ShapeDtypeStruct((NN, DH), jnp.float32),
        jax.ShapeDtypeStruct((NN, DI), jnp.float32),
    ],
)


def _mid_body(aa_ref, ab_ref, ca_ref, cb_ref, s1_ref, wl_ref, wr_ref, b_ref,
              pa_ref, pb_ref, s_ref):
  r = 1.0 / jnp.maximum(ca_ref[:, 0:1] + cb_ref[:, 0:1], 1.0)
  h = jnp.concatenate([aa_ref[:] * r, ab_ref[:] * r], axis=1) + s1_ref[:]
  h = jnp.maximum(h, 0.0)
  p = jnp.dot(h, wl_ref[:], preferred_element_type=jnp.float32)
  pa_ref[:] = p[:, :DH]
  pb_ref[:] = p[:, DH:]
  s_ref[:] = jnp.dot(h, wr_ref[:], preferred_element_type=jnp.float32) + b_ref[:]


_mid = pl.pallas_call(
    _mid_body,
    grid=(NN // RB,),
    in_specs=[
        pl.BlockSpec((RB, DH), lambda i: (i, 0)),
        pl.BlockSpec((RB, DH), lambda i: (i, 0)),
        pl.BlockSpec((RB, DH), lambda i: (i, 0)),
        pl.BlockSpec((RB, DH), lambda i: (i, 0)),
        pl.BlockSpec((RB, DI), lambda i: (i, 0)),
        pl.BlockSpec((DI, DI), lambda i: (0, 0)),
        pl.BlockSpec((DI, DI), lambda i: (0, 0)),
        pl.BlockSpec((1, DI), lambda i: (0, 0)),
    ],
    out_specs=[
        pl.BlockSpec((RB, DH), lambda i: (i, 0)),
        pl.BlockSpec((RB, DH), lambda i: (i, 0)),
        pl.BlockSpec((RB, DI), lambda i: (i, 0)),
    ],
    out_shape=[
        jax.ShapeDtypeStruct((NN, DH), jnp.float32),
        jax.ShapeDtypeStruct((NN, DH), jnp.float32),
        jax.ShapeDtypeStruct((NN, DI), jnp.float32),
    ],
)


def _post_body(aa_ref, ab_ref, ca_ref, cb_ref, s2_ref, wlin_ref, blin_ref,
               out_ref):
  r = 1.0 / jnp.maximum(ca_ref[:, 0:1] + cb_ref[:, 0:1], 1.0)
  h = jnp.concatenate([aa_ref[:] * r, ab_ref[:] * r], axis=1) + s2_ref[:]
  h = jnp.maximum(h, 0.0)
  out_ref[:] = (jnp.dot(h, wlin_ref[:], preferred_element_type=jnp.float32)
                + blin_ref[:])


_post = pl.pallas_call(
    _post_body,
    grid=(NN // RB,),
    in_specs=[
        pl.BlockSpec((RB, DH), lambda i: (i, 0)),
        pl.BlockSpec((RB, DH), lambda i: (i, 0)),
        pl.BlockSpec((RB, DH), lambda i: (i, 0)),
        pl.BlockSpec((RB, DH), lambda i: (i, 0)),
        pl.BlockSpec((RB, DI), lambda i: (i, 0)),
        pl.BlockSpec((DI, DI), lambda i: (0, 0)),
        pl.BlockSpec((1, DI), lambda i: (0, 0)),
    ],
    out_specs=pl.BlockSpec((RB, DI), lambda i: (i, 0)),
    out_shape=jax.ShapeDtypeStruct((NN, DI), jnp.float32),
)


@jax.jit
def kernel(x, edge_index, W1l, b1l, W1r, W2l, b2l, W2r, Wlin, blin):
  src = edge_index[0].astype(jnp.int32)
  dst = edge_index[1].astype(jnp.int32)
  srci = jnp.concatenate(
      [src, jnp.zeros((E_PAD - NE,), jnp.int32)]).reshape(NS, CPT, CH)
  dsti = jnp.concatenate(
      [dst, jnp.full((E_PAD - NE,), NN, jnp.int32)]).reshape(NS, CPT, CH)
  zbig = jnp.zeros((CH, DH), jnp.float32)
  ones2 = jnp.ones((CH, DH), jnp.float32)
  b1 = b1l.reshape(1, DI)
  b2 = b2l.reshape(1, DI)
  bl = blin.reshape(1, DI)

  pa, pb, s1 = _pre(x, W1l, W1r, b1)
  cnta, cntb = _cnt(dsti, zbig, ones2)
  agga, aggb = _agg(pa, pb, srci, dsti, zbig)
  p2a, p2b, s2 = _mid(agga[:NN], aggb[:NN], cnta[:NN], cntb[:NN], s1,
                      W2l, W2r, b2)
  a2a, a2b = _agg(p2a, p2b, srci, dsti, zbig)
  return _post(a2a[:NN], a2b[:NN], cnta[:NN], cntb[:NN], s2, Wlin, bl)


# speedup vs baseline: 1.2309x; 1.0172x over previous
import jax
import jax.numpy as jnp
from jax import lax
from jax.experimental import pallas as pl
from jax.experimental.pallas import tpu as pltpu
from jax.experimental.pallas import tpu_sc as plsc

NN = 10000
NE = 160000
DI = 256
DH = 128
NC = 2
NS = 16
L = 16

CH = 32
CPT = 320
IBK = 8
NIB = CPT // IBK
GB = 64
NBODY = CPT // GB
EPT = CPT * CH
E_PAD = NS * EPT
N_PAD = 10240
ZROWS = N_PAD // NS
ZSTARTS = tuple(range(0, ZROWS, CH))

CW = DH

RB = 1000


def _sc_mesh():
  return plsc.VectorSubcoreMesh(core_axis_name="c", subcore_axis_name="s")


def _edge_pipeline(p_hbm, srci, dsti, s, srcblk, dstblk, rows, sg, ss, acc):
  def blk(b, carry):
    pltpu.sync_copy(srci.at[s, pl.ds(b * GB, GB)], srcblk)
    pltpu.sync_copy(dsti.at[s, pl.ds(b * GB, GB)], dstblk)
    sgh = [None] * 4
    ssh = [None] * 4
    for j in range(GB):
      k = j % 4
      if ssh[k] is not None:
        ssh[k].wait()
      sgh[k] = pltpu.async_copy(p_hbm.at[srcblk.at[j]], rows[k], sg[k])
      if j >= 2:
        k2 = (j - 2) % 4
        sgh[k2].wait()
        ssh[k2] = pltpu.async_copy(rows[k2], acc.at[dstblk.at[j - 2]],
                                   ss[k2], add=True)
    for j in (GB - 2, GB - 1):
      k = j % 4
      sgh[k].wait()
      ssh[k] = pltpu.async_copy(rows[k], acc.at[dstblk.at[j]], ss[k],
                                add=True)
    for k in range(4):
      ssh[k].wait()
    return carry
  lax.fori_loop(0, NBODY, blk, 0)


def _agg_body(pa, pb, srci, dsti, zbig, agga, aggb,
              srcblk, dstblk, rows0, rows1, rows2, rows3, acc,
              sg0, sg1, sg2, sg3, ss0, ss1, ss2, ss3):
  c = lax.axis_index("c")
  s = lax.axis_index("s")
  rows = (rows0, rows1, rows2, rows3)
  sg = (sg0, sg1, sg2, sg3)
  ss = (ss0, ss1, ss2, ss3)

  pltpu.sync_copy(zbig, rows0)
  for off in ZSTARTS:
    pltpu.sync_copy(rows0, acc.at[pl.ds(s * ZROWS + off, CH)])
  plsc.subcore_barrier()

  @pl.when(c == 0)
  def _():
    _edge_pipeline(pa, srci, dsti, s, srcblk, dstblk, rows, sg, ss, acc)

  @pl.when(c == 1)
  def _():
    _edge_pipeline(pb, srci, dsti, s, srcblk, dstblk, rows, sg, ss, acc)

  plsc.subcore_barrier()

  @pl.when(c == 0)
  def _():
    for off in ZSTARTS:
      base = s * ZROWS + off
      pltpu.sync_copy(acc.at[pl.ds(base, CH)], rows0)
      pltpu.sync_copy(rows0, agga.at[pl.ds(base, CH)])

  @pl.when(c == 1)
  def _():
    for off in ZSTARTS:
      base = s * ZROWS + off
      pltpu.sync_copy(acc.at[pl.ds(base, CH)], rows0)
      pltpu.sync_copy(rows0, aggb.at[pl.ds(base, CH)])


_agg = pl.kernel(
    _agg_body,
    out_type=(
        jax.ShapeDtypeStruct((N_PAD, DH), jnp.float32),
        jax.ShapeDtypeStruct((N_PAD, DH), jnp.float32),
    ),
    mesh=_sc_mesh(),
    scratch_types=[
        pltpu.VMEM((GB, CH), jnp.int32),
        pltpu.VMEM((GB, CH), jnp.int32),
        pltpu.VMEM((CH, DH), jnp.float32),
        pltpu.VMEM((CH, DH), jnp.float32),
        pltpu.VMEM((CH, DH), jnp.float32),
        pltpu.VMEM((CH, DH), jnp.float32),
        pltpu.VMEM_SHARED((N_PAD, DH), jnp.float32),
        pltpu.SemaphoreType.DMA,
        pltpu.SemaphoreType.DMA,
        pltpu.SemaphoreType.DMA,
        pltpu.SemaphoreType.DMA,
        pltpu.SemaphoreType.DMA,
        pltpu.SemaphoreType.DMA,
        pltpu.SemaphoreType.DMA,
        pltpu.SemaphoreType.DMA,
    ],
)


CPT0 = CPT // 2


def _cnt_body(dsti, zc32, on32, cnta, cntb, dstblk, buf, acc, sem):
  c = lax.axis_index("c")
  s = lax.axis_index("s")

  pltpu.sync_copy(zc32, buf)
  for off in ZSTARTS:
    pltpu.sync_copy(buf, acc.at[pl.ds(s * ZROWS + off, CH)])
  pltpu.sync_copy(on32, buf)
  plsc.subcore_barrier()

  def blk(b, carry):
    pltpu.sync_copy(dsti.at[s, pl.ds(b * IBK, IBK)], dstblk)
    for k in range(IBK):
      pltpu.sync_copy(buf, acc.at[dstblk.at[k]], add=True)
    return carry

  @pl.when(c == 0)
  def _():
    lax.fori_loop(0, CPT0 // IBK, blk, 0)

  @pl.when(c == 1)
  def _():
    lax.fori_loop(CPT0 // IBK, NIB, blk, 0)

  plsc.subcore_barrier()

  @pl.when(c == 0)
  def _():
    for off in ZSTARTS:
      base = s * ZROWS + off
      pltpu.sync_copy(acc.at[pl.ds(base, CH)], buf)
      pltpu.sync_copy(buf, cnta.at[pl.ds(base, CH)])

  @pl.when(c == 1)
  def _():
    for off in ZSTARTS:
      base = s * ZROWS + off
      pltpu.sync_copy(acc.at[pl.ds(base, CH)], buf)
      pltpu.sync_copy(buf, cntb.at[pl.ds(base, CH)])


_cnt = pl.kernel(
    _cnt_body,
    out_type=(
        jax.ShapeDtypeStruct((N_PAD, CW), jnp.float32),
        jax.ShapeDtypeStruct((N_PAD, CW), jnp.float32),
    ),
    mesh=_sc_mesh(),
    scratch_types=[
        pltpu.VMEM((IBK, CH), jnp.int32),
        pltpu.VMEM((CH, CW), jnp.float32),
        pltpu.VMEM_SHARED((N_PAD, CW), jnp.float32),
        pltpu.SemaphoreType.DMA,
    ],
)


def _pre_body(x_ref, wl_ref, wr_ref, b_ref, pa_ref, pb_ref, s_ref):
  xb = x_ref[:]
  p = jnp.dot(xb, wl_ref[:], preferred_element_type=jnp.float32)
  pa_ref[:] = p[:, :DH]
  pb_ref[:] = p[:, DH:]
  s_ref[:] = jnp.dot(xb, wr_ref[:], preferred_element_type=jnp.float32) + b_ref[:]


_pre = pl.pallas_call(
    _pre_body,
    grid=(NN // RB,),
    in_specs=[
        pl.BlockSpec((RB, DI), lambda i: (i, 0)),
        pl.BlockSpec((DI, DI), lambda i: (0, 0)),
        pl.BlockSpec((DI, DI), lambda i: (0, 0)),
        pl.BlockSpec((1, DI), lambda i: (0, 0)),
    ],
    out_specs=[
        pl.BlockSpec((RB, DH), lambda i: (i, 0)),
        pl.BlockSpec((RB, DH), lambda i: (i, 0)),
        pl.BlockSpec((RB, DI), lambda i: (i, 0)),
    ],
    out_shape=[
        jax.ShapeDtypeStruct((NN, DH), jnp.float32),
        jax.ShapeDtypeStruct((NN, DH), jnp.float32),
        jax.ShapeDtypeStruct((NN, DI), jnp.float32),
    ],
)


def _mid_body(aa_ref, ab_ref, ca_ref, cb_ref, s1_ref, wl_ref, wr_ref, b_ref,
              pa_ref, pb_ref, s_ref):
  r = 1.0 / jnp.maximum(ca_ref[:, 0:1] + cb_ref[:, 0:1], 1.0)
  h = jnp.concatenate([aa_ref[:] * r, ab_ref[:] * r], axis=1) + s1_ref[:]
  h = jnp.maximum(h, 0.0)
  p = jnp.dot(h, wl_ref[:], preferred_element_type=jnp.float32)
  pa_ref[:] = p[:, :DH]
  pb_ref[:] = p[:, DH:]
  s_ref[:] = jnp.dot(h, wr_ref[:], preferred_element_type=jnp.float32) + b_ref[:]


_mid = pl.pallas_call(
    _mid_body,
    grid=(NN // RB,),
    in_specs=[
        pl.BlockSpec((RB, DH), lambda i: (i, 0)),
        pl.BlockSpec((RB, DH), lambda i: (i, 0)),
        pl.BlockSpec((RB, CW), lambda i: (i, 0)),
        pl.BlockSpec((RB, CW), lambda i: (i, 0)),
        pl.BlockSpec((RB, DI), lambda i: (i, 0)),
        pl.BlockSpec((DI, DI), lambda i: (0, 0)),
        pl.BlockSpec((DI, DI), lambda i: (0, 0)),
        pl.BlockSpec((1, DI), lambda i: (0, 0)),
    ],
    out_specs=[
        pl.BlockSpec((RB, DH), lambda i: (i, 0)),
        pl.BlockSpec((RB, DH), lambda i: (i, 0)),
        pl.BlockSpec((RB, DI), lambda i: (i, 0)),
    ],
    out_shape=[
        jax.ShapeDtypeStruct((NN, DH), jnp.float32),
        jax.ShapeDtypeStruct((NN, DH), jnp.float32),
        jax.ShapeDtypeStruct((NN, DI), jnp.float32),
    ],
)


def _post_body(aa_ref, ab_ref, ca_ref, cb_ref, s2_ref, wlin_ref, blin_ref,
               out_ref):
  r = 1.0 / jnp.maximum(ca_ref[:, 0:1] + cb_ref[:, 0:1], 1.0)
  h = jnp.concatenate([aa_ref[:] * r, ab_ref[:] * r], axis=1) + s2_ref[:]
  h = jnp.maximum(h, 0.0)
  out_ref[:] = (jnp.dot(h, wlin_ref[:], preferred_element_type=jnp.float32)
                + blin_ref[:])


_post = pl.pallas_call(
    _post_body,
    grid=(NN // RB,),
    in_specs=[
        pl.BlockSpec((RB, DH), lambda i: (i, 0)),
        pl.BlockSpec((RB, DH), lambda i: (i, 0)),
        pl.BlockSpec((RB, CW), lambda i: (i, 0)),
        pl.BlockSpec((RB, CW), lambda i: (i, 0)),
        pl.BlockSpec((RB, DI), lambda i: (i, 0)),
        pl.BlockSpec((DI, DI), lambda i: (0, 0)),
        pl.BlockSpec((1, DI), lambda i: (0, 0)),
    ],
    out_specs=pl.BlockSpec((RB, DI), lambda i: (i, 0)),
    out_shape=jax.ShapeDtypeStruct((NN, DI), jnp.float32),
)


@jax.jit
def kernel(x, edge_index, W1l, b1l, W1r, W2l, b2l, W2r, Wlin, blin):
  src = edge_index[0].astype(jnp.int32)
  dst = edge_index[1].astype(jnp.int32)
  srci = jnp.concatenate(
      [src, jnp.zeros((E_PAD - NE,), jnp.int32)]).reshape(NS, CPT, CH)
  dsti = jnp.concatenate(
      [dst, jnp.full((E_PAD - NE,), NN, jnp.int32)]).reshape(NS, CPT, CH)
  zbig = jnp.zeros((CH, DH), jnp.float32)
  zc32 = jnp.zeros((CH, CW), jnp.float32)
  on32 = jnp.ones((CH, CW), jnp.float32)
  b1 = b1l.reshape(1, DI)
  b2 = b2l.reshape(1, DI)
  bl = blin.reshape(1, DI)

  pa, pb, s1 = _pre(x, W1l, W1r, b1)
  cnta, cntb = _cnt(dsti, zc32, on32)
  agga, aggb = _agg(pa, pb, srci, dsti, zbig)
  p2a, p2b, s2 = _mid(agga[:NN], aggb[:NN], cnta[:NN], cntb[:NN], s1,
                      W2l, W2r, b2)
  a2a, a2b = _agg(p2a, p2b, srci, dsti, zbig)
  return _post(a2a[:NN], a2b[:NN], cnta[:NN], cntb[:NN], s2, Wlin, bl)


# speedup vs baseline: 1.2555x; 1.0200x over previous
import jax
import jax.numpy as jnp
from jax import lax
from jax.experimental import pallas as pl
from jax.experimental.pallas import tpu as pltpu
from jax.experimental.pallas import tpu_sc as plsc

NN = 10000
NE = 160000
DI = 256
DH = 128
NC = 2
NS = 16
L = 16

CH = 32
CPT = 320
IBK = 32
NIB = CPT // IBK
GB = 64
NBODY = CPT // GB
EPT = CPT * CH
E_PAD = NS * EPT
N_PAD = 10240
ZROWS = N_PAD // NS
ZSTARTS = tuple(range(0, ZROWS, CH))

CW = DH

RB = 2000


def _sc_mesh():
  return plsc.VectorSubcoreMesh(core_axis_name="c", subcore_axis_name="s")


def _edge_pipeline(p_hbm, srci, dsti, s, srcblk, dstblk, rows, sg, ss, acc):
  def blk(b, carry):
    pltpu.sync_copy(srci.at[s, pl.ds(b * GB, GB)], srcblk)
    pltpu.sync_copy(dsti.at[s, pl.ds(b * GB, GB)], dstblk)
    sgh = [None] * 4
    ssh = [None] * 4
    for j in range(GB):
      k = j % 4
      if ssh[k] is not None:
        ssh[k].wait()
      sgh[k] = pltpu.async_copy(p_hbm.at[srcblk.at[j]], rows[k], sg[k])
      if j >= 2:
        k2 = (j - 2) % 4
        sgh[k2].wait()
        ssh[k2] = pltpu.async_copy(rows[k2], acc.at[dstblk.at[j - 2]],
                                   ss[k2], add=True)
    for j in (GB - 2, GB - 1):
      k = j % 4
      sgh[k].wait()
      ssh[k] = pltpu.async_copy(rows[k], acc.at[dstblk.at[j]], ss[k],
                                add=True)
    for k in range(4):
      ssh[k].wait()
    return carry
  lax.fori_loop(0, NBODY, blk, 0)


def _agg_body(pa, pb, srci, dsti, zbig, agga, aggb,
              srcblk, dstblk, rows0, rows1, rows2, rows3, acc,
              sg0, sg1, sg2, sg3, ss0, ss1, ss2, ss3):
  c = lax.axis_index("c")
  s = lax.axis_index("s")
  rows = (rows0, rows1, rows2, rows3)
  sg = (sg0, sg1, sg2, sg3)
  ss = (ss0, ss1, ss2, ss3)

  pltpu.sync_copy(zbig, rows0)
  for off in ZSTARTS:
    pltpu.sync_copy(rows0, acc.at[pl.ds(s * ZROWS + off, CH)])
  plsc.subcore_barrier()

  @pl.when(c == 0)
  def _():
    _edge_pipeline(pa, srci, dsti, s, srcblk, dstblk, rows, sg, ss, acc)

  @pl.when(c == 1)
  def _():
    _edge_pipeline(pb, srci, dsti, s, srcblk, dstblk, rows, sg, ss, acc)

  plsc.subcore_barrier()

  @pl.when(c == 0)
  def _():
    for off in ZSTARTS:
      base = s * ZROWS + off
      pltpu.sync_copy(acc.at[pl.ds(base, CH)], rows0)
      pltpu.sync_copy(rows0, agga.at[pl.ds(base, CH)])

  @pl.when(c == 1)
  def _():
    for off in ZSTARTS:
      base = s * ZROWS + off
      pltpu.sync_copy(acc.at[pl.ds(base, CH)], rows0)
      pltpu.sync_copy(rows0, aggb.at[pl.ds(base, CH)])


_agg = pl.kernel(
    _agg_body,
    out_type=(
        jax.ShapeDtypeStruct((N_PAD, DH), jnp.float32),
        jax.ShapeDtypeStruct((N_PAD, DH), jnp.float32),
    ),
    mesh=_sc_mesh(),
    scratch_types=[
        pltpu.VMEM((GB, CH), jnp.int32),
        pltpu.VMEM((GB, CH), jnp.int32),
        pltpu.VMEM((CH, DH), jnp.float32),
        pltpu.VMEM((CH, DH), jnp.float32),
        pltpu.VMEM((CH, DH), jnp.float32),
        pltpu.VMEM((CH, DH), jnp.float32),
        pltpu.VMEM_SHARED((N_PAD, DH), jnp.float32),
        pltpu.SemaphoreType.DMA,
        pltpu.SemaphoreType.DMA,
        pltpu.SemaphoreType.DMA,
        pltpu.SemaphoreType.DMA,
        pltpu.SemaphoreType.DMA,
        pltpu.SemaphoreType.DMA,
        pltpu.SemaphoreType.DMA,
        pltpu.SemaphoreType.DMA,
    ],
)


CPT0 = CPT // 2


def _cnt_body(dsti, zc32, on32, cnta, cntb, dstblk, buf, acc, sem):
  c = lax.axis_index("c")
  s = lax.axis_index("s")

  pltpu.sync_copy(zc32, buf)
  for off in ZSTARTS:
    pltpu.sync_copy(buf, acc.at[pl.ds(s * ZROWS + off, CH)])
  pltpu.sync_copy(on32, buf)
  plsc.subcore_barrier()

  def blk(b, carry):
    pltpu.sync_copy(dsti.at[s, pl.ds(b * IBK, IBK)], dstblk)
    hs = [pltpu.async_copy(buf, acc.at[dstblk.at[k]], sem, add=True)
          for k in range(IBK)]
    for h in hs:
      h.wait()
    return carry

  @pl.when(c == 0)
  def _():
    lax.fori_loop(0, CPT0 // IBK, blk, 0)

  @pl.when(c == 1)
  def _():
    lax.fori_loop(CPT0 // IBK, NIB, blk, 0)

  plsc.subcore_barrier()

  @pl.when(c == 0)
  def _():
    for off in ZSTARTS:
      base = s * ZROWS + off
      pltpu.sync_copy(acc.at[pl.ds(base, CH)], buf)
      pltpu.sync_copy(buf, cnta.at[pl.ds(base, CH)])

  @pl.when(c == 1)
  def _():
    for off in ZSTARTS:
      base = s * ZROWS + off
      pltpu.sync_copy(acc.at[pl.ds(base, CH)], buf)
      pltpu.sync_copy(buf, cntb.at[pl.ds(base, CH)])


_cnt = pl.kernel(
    _cnt_body,
    out_type=(
        jax.ShapeDtypeStruct((N_PAD, CW), jnp.float32),
        jax.ShapeDtypeStruct((N_PAD, CW), jnp.float32),
    ),
    mesh=_sc_mesh(),
    scratch_types=[
        pltpu.VMEM((IBK, CH), jnp.int32),
        pltpu.VMEM((CH, CW), jnp.float32),
        pltpu.VMEM_SHARED((N_PAD, CW), jnp.float32),
        pltpu.SemaphoreType.DMA,
    ],
)


def _pre_body(x_ref, wl_ref, wr_ref, b_ref, pa_ref, pb_ref, s_ref):
  xb = x_ref[:]
  p = jnp.dot(xb, wl_ref[:], preferred_element_type=jnp.float32)
  pa_ref[:] = p[:, :DH]
  pb_ref[:] = p[:, DH:]
  s_ref[:] = jnp.dot(xb, wr_ref[:], preferred_element_type=jnp.float32) + b_ref[:]


_pre = pl.pallas_call(
    _pre_body,
    grid=(NN // RB,),
    in_specs=[
        pl.BlockSpec((RB, DI), lambda i: (i, 0)),
        pl.BlockSpec((DI, DI), lambda i: (0, 0)),
        pl.BlockSpec((DI, DI), lambda i: (0, 0)),
        pl.BlockSpec((1, DI), lambda i: (0, 0)),
    ],
    out_specs=[
        pl.BlockSpec((RB, DH), lambda i: (i, 0)),
        pl.BlockSpec((RB, DH), lambda i: (i, 0)),
        pl.BlockSpec((RB, DI), lambda i: (i, 0)),
    ],
    out_shape=[
        jax.ShapeDtypeStruct((NN, DH), jnp.float32),
        jax.ShapeDtypeStruct((NN, DH), jnp.float32),
        jax.ShapeDtypeStruct((NN, DI), jnp.float32),
    ],
)


def _mid_body(aa_ref, ab_ref, ca_ref, cb_ref, s1_ref, wl_ref, wr_ref, b_ref,
              pa_ref, pb_ref, s_ref):
  r = 1.0 / jnp.maximum(ca_ref[:, 0:1] + cb_ref[:, 0:1], 1.0)
  h = jnp.concatenate([aa_ref[:] * r, ab_ref[:] * r], axis=1) + s1_ref[:]
  h = jnp.maximum(h, 0.0)
  p = jnp.dot(h, wl_ref[:], preferred_element_type=jnp.float32)
  pa_ref[:] = p[:, :DH]
  pb_ref[:] = p[:, DH:]
  s_ref[:] = jnp.dot(h, wr_ref[:], preferred_element_type=jnp.float32) + b_ref[:]


_mid = pl.pallas_call(
    _mid_body,
    grid=(NN // RB,),
    in_specs=[
        pl.BlockSpec((RB, DH), lambda i: (i, 0)),
        pl.BlockSpec((RB, DH), lambda i: (i, 0)),
        pl.BlockSpec((RB, CW), lambda i: (i, 0)),
        pl.BlockSpec((RB, CW), lambda i: (i, 0)),
        pl.BlockSpec((RB, DI), lambda i: (i, 0)),
        pl.BlockSpec((DI, DI), lambda i: (0, 0)),
        pl.BlockSpec((DI, DI), lambda i: (0, 0)),
        pl.BlockSpec((1, DI), lambda i: (0, 0)),
    ],
    out_specs=[
        pl.BlockSpec((RB, DH), lambda i: (i, 0)),
        pl.BlockSpec((RB, DH), lambda i: (i, 0)),
        pl.BlockSpec((RB, DI), lambda i: (i, 0)),
    ],
    out_shape=[
        jax.ShapeDtypeStruct((NN, DH), jnp.float32),
        jax.ShapeDtypeStruct((NN, DH), jnp.float32),
        jax.ShapeDtypeStruct((NN, DI), jnp.float32),
    ],
)


def _post_body(aa_ref, ab_ref, ca_ref, cb_ref, s2_ref, wlin_ref, blin_ref,
               out_ref):
  r = 1.0 / jnp.maximum(ca_ref[:, 0:1] + cb_ref[:, 0:1], 1.0)
  h = jnp.concatenate([aa_ref[:] * r, ab_ref[:] * r], axis=1) + s2_ref[:]
  h = jnp.maximum(h, 0.0)
  out_ref[:] = (jnp.dot(h, wlin_ref[:], preferred_element_type=jnp.float32)
                + blin_ref[:])


_post = pl.pallas_call(
    _post_body,
    grid=(NN // RB,),
    in_specs=[
        pl.BlockSpec((RB, DH), lambda i: (i, 0)),
        pl.BlockSpec((RB, DH), lambda i: (i, 0)),
        pl.BlockSpec((RB, CW), lambda i: (i, 0)),
        pl.BlockSpec((RB, CW), lambda i: (i, 0)),
        pl.BlockSpec((RB, DI), lambda i: (i, 0)),
        pl.BlockSpec((DI, DI), lambda i: (0, 0)),
        pl.BlockSpec((1, DI), lambda i: (0, 0)),
    ],
    out_specs=pl.BlockSpec((RB, DI), lambda i: (i, 0)),
    out_shape=jax.ShapeDtypeStruct((NN, DI), jnp.float32),
)


@jax.jit
def kernel(x, edge_index, W1l, b1l, W1r, W2l, b2l, W2r, Wlin, blin):
  src = edge_index[0].astype(jnp.int32)
  dst = edge_index[1].astype(jnp.int32)
  srci = jnp.concatenate(
      [src, jnp.zeros((E_PAD - NE,), jnp.int32)]).reshape(NS, CPT, CH)
  dsti = jnp.concatenate(
      [dst, jnp.full((E_PAD - NE,), NN, jnp.int32)]).reshape(NS, CPT, CH)
  zbig = jnp.zeros((CH, DH), jnp.float32)
  zc32 = jnp.zeros((CH, CW), jnp.float32)
  on32 = jnp.ones((CH, CW), jnp.float32)
  b1 = b1l.reshape(1, DI)
  b2 = b2l.reshape(1, DI)
  bl = blin.reshape(1, DI)

  pa, pb, s1 = _pre(x, W1l, W1r, b1)
  cnta, cntb = _cnt(dsti, zc32, on32)
  agga, aggb = _agg(pa, pb, srci, dsti, zbig)
  p2a, p2b, s2 = _mid(agga[:NN], aggb[:NN], cnta[:NN], cntb[:NN], s1,
                      W2l, W2r, b2)
  a2a, a2b = _agg(p2a, p2b, srci, dsti, zbig)
  return _post(a2a[:NN], a2b[:NN], cnta[:NN], cntb[:NN], s2, Wlin, bl)


# speedup vs baseline: 1.2748x; 1.0154x over previous
import jax
import jax.numpy as jnp
from jax import lax
from jax.experimental import pallas as pl
from jax.experimental.pallas import tpu as pltpu
from jax.experimental.pallas import tpu_sc as plsc

NN = 10000
NE = 160000
DI = 256
DH = 128
NC = 2
NS = 16
L = 16

CH = 32
CPT = 320
IBK = 32
NIB = CPT // IBK
GB = 64
NBODY = CPT // GB
EPT = CPT * CH
E_PAD = NS * EPT
N_PAD = 10240
ZROWS = N_PAD // NS
ZSTARTS = tuple(range(0, ZROWS, CH))

CW = DH

RB = 2000


def _sc_mesh():
  return plsc.VectorSubcoreMesh(core_axis_name="c", subcore_axis_name="s")


def _edge_pipeline(p_hbm, srci, dsti, s, srcblk, dstblk, rows, sg, ss, acc):
  def blk(b, carry):
    pltpu.sync_copy(srci.at[s, pl.ds(b * GB, GB)], srcblk)
    pltpu.sync_copy(dsti.at[s, pl.ds(b * GB, GB)], dstblk)
    sgh = [None] * 4
    ssh = [None] * 4
    for j in range(GB):
      k = j % 4
      if ssh[k] is not None:
        ssh[k].wait()
      sgh[k] = pltpu.async_copy(p_hbm.at[srcblk.at[j]], rows[k], sg[k])
      if j >= 2:
        k2 = (j - 2) % 4
        sgh[k2].wait()
        ssh[k2] = pltpu.async_copy(rows[k2], acc.at[dstblk.at[j - 2]],
                                   ss[k2], add=True)
    for j in (GB - 2, GB - 1):
      k = j % 4
      sgh[k].wait()
      ssh[k] = pltpu.async_copy(rows[k], acc.at[dstblk.at[j]], ss[k],
                                add=True)
    for k in range(4):
      ssh[k].wait()
    return carry
  lax.fori_loop(0, NBODY, blk, 0)


def _agg_body(pa, pb, srci, dsti, zbig, agga, aggb,
              srcblk, dstblk, rows0, rows1, rows2, rows3, acc,
              sg0, sg1, sg2, sg3, ss0, ss1, ss2, ss3):
  c = lax.axis_index("c")
  s = lax.axis_index("s")
  rows = (rows0, rows1, rows2, rows3)
  sg = (sg0, sg1, sg2, sg3)
  ss = (ss0, ss1, ss2, ss3)

  pltpu.sync_copy(zbig, rows0)
  zh = [pltpu.async_copy(rows0, acc.at[pl.ds(s * ZROWS + off, CH)], sg0)
        for off in ZSTARTS]
  for h in zh:
    h.wait()
  plsc.subcore_barrier()

  @pl.when(c == 0)
  def _():
    _edge_pipeline(pa, srci, dsti, s, srcblk, dstblk, rows, sg, ss, acc)

  @pl.when(c == 1)
  def _():
    _edge_pipeline(pb, srci, dsti, s, srcblk, dstblk, rows, sg, ss, acc)

  plsc.subcore_barrier()

  def copy_out(out_ref):
    outh = [None] * 4
    prev = None
    for i, off in enumerate(ZSTARTS):
      b = i % 4
      base = s * ZROWS + off
      if outh[b] is not None:
        outh[b].wait()
      h = pltpu.async_copy(acc.at[pl.ds(base, CH)], rows[b], sg[b])
      if prev is not None:
        ph, pb, pbase = prev
        ph.wait()
        outh[pb] = pltpu.async_copy(rows[pb], out_ref.at[pl.ds(pbase, CH)],
                                    ss[pb])
      prev = (h, b, base)
    ph, pb, pbase = prev
    ph.wait()
    outh[pb] = pltpu.async_copy(rows[pb], out_ref.at[pl.ds(pbase, CH)],
                                ss[pb])
    for h in outh:
      if h is not None:
        h.wait()

  @pl.when(c == 0)
  def _():
    copy_out(agga)

  @pl.when(c == 1)
  def _():
    copy_out(aggb)


_agg = pl.kernel(
    _agg_body,
    out_type=(
        jax.ShapeDtypeStruct((N_PAD, DH), jnp.float32),
        jax.ShapeDtypeStruct((N_PAD, DH), jnp.float32),
    ),
    mesh=_sc_mesh(),
    scratch_types=[
        pltpu.VMEM((GB, CH), jnp.int32),
        pltpu.VMEM((GB, CH), jnp.int32),
        pltpu.VMEM((CH, DH), jnp.float32),
        pltpu.VMEM((CH, DH), jnp.float32),
        pltpu.VMEM((CH, DH), jnp.float32),
        pltpu.VMEM((CH, DH), jnp.float32),
        pltpu.VMEM_SHARED((N_PAD, DH), jnp.float32),
        pltpu.SemaphoreType.DMA,
        pltpu.SemaphoreType.DMA,
        pltpu.SemaphoreType.DMA,
        pltpu.SemaphoreType.DMA,
        pltpu.SemaphoreType.DMA,
        pltpu.SemaphoreType.DMA,
        pltpu.SemaphoreType.DMA,
        pltpu.SemaphoreType.DMA,
    ],
)


CPT0 = CPT // 2


def _cnt_body(dsti, zc32, on32, cnta, cntb, dstblk, buf, acc, sem):
  c = lax.axis_index("c")
  s = lax.axis_index("s")

  pltpu.sync_copy(zc32, buf)
  for off in ZSTARTS:
    pltpu.sync_copy(buf, acc.at[pl.ds(s * ZROWS + off, CH)])
  pltpu.sync_copy(on32, buf)
  plsc.subcore_barrier()

  def blk(b, carry):
    pltpu.sync_copy(dsti.at[s, pl.ds(b * IBK, IBK)], dstblk)
    hs = [pltpu.async_copy(buf, acc.at[dstblk.at[k]], sem, add=True)
          for k in range(IBK)]
    for h in hs:
      h.wait()
    return carry

  @pl.when(c == 0)
  def _():
    lax.fori_loop(0, CPT0 // IBK, blk, 0)

  @pl.when(c == 1)
  def _():
    lax.fori_loop(CPT0 // IBK, NIB, blk, 0)

  plsc.subcore_barrier()

  @pl.when(c == 0)
  def _():
    for off in ZSTARTS:
      base = s * ZROWS + off
      pltpu.sync_copy(acc.at[pl.ds(base, CH)], buf)
      pltpu.sync_copy(buf, cnta.at[pl.ds(base, CH)])

  @pl.when(c == 1)
  def _():
    for off in ZSTARTS:
      base = s * ZROWS + off
      pltpu.sync_copy(acc.at[pl.ds(base, CH)], buf)
      pltpu.sync_copy(buf, cntb.at[pl.ds(base, CH)])


_cnt = pl.kernel(
    _cnt_body,
    out_type=(
        jax.ShapeDtypeStruct((N_PAD, CW), jnp.float32),
        jax.ShapeDtypeStruct((N_PAD, CW), jnp.float32),
    ),
    mesh=_sc_mesh(),
    scratch_types=[
        pltpu.VMEM((IBK, CH), jnp.int32),
        pltpu.VMEM((CH, CW), jnp.float32),
        pltpu.VMEM_SHARED((N_PAD, CW), jnp.float32),
        pltpu.SemaphoreType.DMA,
    ],
)


def _pre_body(x_ref, wl_ref, wr_ref, b_ref, pa_ref, pb_ref, s_ref):
  xb = x_ref[:]
  p = jnp.dot(xb, wl_ref[:], preferred_element_type=jnp.float32)
  pa_ref[:] = p[:, :DH]
  pb_ref[:] = p[:, DH:]
  s_ref[:] = jnp.dot(xb, wr_ref[:], preferred_element_type=jnp.float32) + b_ref[:]


_pre = pl.pallas_call(
    _pre_body,
    grid=(NN // RB,),
    in_specs=[
        pl.BlockSpec((RB, DI), lambda i: (i, 0)),
        pl.BlockSpec((DI, DI), lambda i: (0, 0)),
        pl.BlockSpec((DI, DI), lambda i: (0, 0)),
        pl.BlockSpec((1, DI), lambda i: (0, 0)),
    ],
    out_specs=[
        pl.BlockSpec((RB, DH), lambda i: (i, 0)),
        pl.BlockSpec((RB, DH), lambda i: (i, 0)),
        pl.BlockSpec((RB, DI), lambda i: (i, 0)),
    ],
    out_shape=[
        jax.ShapeDtypeStruct((NN, DH), jnp.float32),
        jax.ShapeDtypeStruct((NN, DH), jnp.float32),
        jax.ShapeDtypeStruct((NN, DI), jnp.float32),
    ],
)


def _mid_body(aa_ref, ab_ref, ca_ref, cb_ref, s1_ref, wl_ref, wr_ref, b_ref,
              pa_ref, pb_ref, s_ref):
  r = 1.0 / jnp.maximum(ca_ref[:, 0:1] + cb_ref[:, 0:1], 1.0)
  h = jnp.concatenate([aa_ref[:] * r, ab_ref[:] * r], axis=1) + s1_ref[:]
  h = jnp.maximum(h, 0.0)
  p = jnp.dot(h, wl_ref[:], preferred_element_type=jnp.float32)
  pa_ref[:] = p[:, :DH]
  pb_ref[:] = p[:, DH:]
  s_ref[:] = jnp.dot(h, wr_ref[:], preferred_element_type=jnp.float32) + b_ref[:]


_mid = pl.pallas_call(
    _mid_body,
    grid=(NN // RB,),
    in_specs=[
        pl.BlockSpec((RB, DH), lambda i: (i, 0)),
        pl.BlockSpec((RB, DH), lambda i: (i, 0)),
        pl.BlockSpec((RB, CW), lambda i: (i, 0)),
        pl.BlockSpec((RB, CW), lambda i: (i, 0)),
        pl.BlockSpec((RB, DI), lambda i: (i, 0)),
        pl.BlockSpec((DI, DI), lambda i: (0, 0)),
        pl.BlockSpec((DI, DI), lambda i: (0, 0)),
        pl.BlockSpec((1, DI), lambda i: (0, 0)),
    ],
    out_specs=[
        pl.BlockSpec((RB, DH), lambda i: (i, 0)),
        pl.BlockSpec((RB, DH), lambda i: (i, 0)),
        pl.BlockSpec((RB, DI), lambda i: (i, 0)),
    ],
    out_shape=[
        jax.ShapeDtypeStruct((NN, DH), jnp.float32),
        jax.ShapeDtypeStruct((NN, DH), jnp.float32),
        jax.ShapeDtypeStruct((NN, DI), jnp.float32),
    ],
)


def _post_body(aa_ref, ab_ref, ca_ref, cb_ref, s2_ref, wlin_ref, blin_ref,
               out_ref):
  r = 1.0 / jnp.maximum(ca_ref[:, 0:1] + cb_ref[:, 0:1], 1.0)
  h = jnp.concatenate([aa_ref[:] * r, ab_ref[:] * r], axis=1) + s2_ref[:]
  h = jnp.maximum(h, 0.0)
  out_ref[:] = (jnp.dot(h, wlin_ref[:], preferred_element_type=jnp.float32)
                + blin_ref[:])


_post = pl.pallas_call(
    _post_body,
    grid=(NN // RB,),
    in_specs=[
        pl.BlockSpec((RB, DH), lambda i: (i, 0)),
        pl.BlockSpec((RB, DH), lambda i: (i, 0)),
        pl.BlockSpec((RB, CW), lambda i: (i, 0)),
        pl.BlockSpec((RB, CW), lambda i: (i, 0)),
        pl.BlockSpec((RB, DI), lambda i: (i, 0)),
        pl.BlockSpec((DI, DI), lambda i: (0, 0)),
        pl.BlockSpec((1, DI), lambda i: (0, 0)),
    ],
    out_specs=pl.BlockSpec((RB, DI), lambda i: (i, 0)),
    out_shape=jax.ShapeDtypeStruct((NN, DI), jnp.float32),
)


@jax.jit
def kernel(x, edge_index, W1l, b1l, W1r, W2l, b2l, W2r, Wlin, blin):
  src = edge_index[0].astype(jnp.int32)
  dst = edge_index[1].astype(jnp.int32)
  srci = jnp.concatenate(
      [src, jnp.zeros((E_PAD - NE,), jnp.int32)]).reshape(NS, CPT, CH)
  dsti = jnp.concatenate(
      [dst, jnp.full((E_PAD - NE,), NN, jnp.int32)]).reshape(NS, CPT, CH)
  zbig = jnp.zeros((CH, DH), jnp.float32)
  zc32 = jnp.zeros((CH, CW), jnp.float32)
  on32 = jnp.ones((CH, CW), jnp.float32)
  b1 = b1l.reshape(1, DI)
  b2 = b2l.reshape(1, DI)
  bl = blin.reshape(1, DI)

  pa, pb, s1 = _pre(x, W1l, W1r, b1)
  cnta, cntb = _cnt(dsti, zc32, on32)
  agga, aggb = _agg(pa, pb, srci, dsti, zbig)
  p2a, p2b, s2 = _mid(agga[:NN], aggb[:NN], cnta[:NN], cntb[:NN], s1,
                      W2l, W2r, b2)
  a2a, a2b = _agg(p2a, p2b, srci, dsti, zbig)
  return _post(a2a[:NN], a2b[:NN], cnta[:NN], cntb[:NN], s2, Wlin, bl)
